# baseline (speedup 1.0000x reference)
import functools

import jax
import jax.numpy as jnp
from jax import lax
from jax.experimental import pallas as pl
from jax.experimental.pallas import tpu as pltpu
from jax.experimental.pallas import tpu_sc as plsc

_NC, _NS = 2, 16
_NW = _NC * _NS


def _vq_idx_body(z_ref, cbT_ref, cb_ref, idx_ref):
    zb = z_ref[...]
    cbT = cbT_ref[...]
    cb = cb_ref[...]
    K = cb.shape[0]
    BT = zb.shape[0]
    cross = lax.dot_general(
        zb, cbT, (((1,), (0,)), ((), ())),
        preferred_element_type=jnp.float32)
    z_sq = jnp.sum(zb * zb, axis=-1, keepdims=True)
    e_sq = jnp.sum(cb * cb, axis=-1)
    dists = z_sq - 2.0 * cross + e_sq[None, :]
    m = jnp.min(dists, axis=-1, keepdims=True)
    mask_f = jnp.where(dists == m, 1.0, 0.0)
    kcol = lax.broadcasted_iota(jnp.int32, (K, 128), 0)
    lcol = lax.broadcasted_iota(jnp.int32, (K, 128), 1)
    ext = jnp.where(lcol == 0, kcol // 32,
                    jnp.where(lcol == 1, kcol % 32, 1)).astype(jnp.float32)
    r = lax.dot_general(
        mask_f, ext, (((1,), (0,)), ((), ())),
        preferred_element_type=jnp.float32)
    hi = r[:, 0].astype(jnp.int32)
    lo = r[:, 1].astype(jnp.int32)
    cnt = r[:, 2]
    idx_ref[...] = hi * 32 + lo
    has_tie = jnp.max(cnt) > 1.0
    @pl.when(has_tie)
    def _():
        kiota = lax.broadcasted_iota(jnp.int32, (BT, K), 1)
        idx_ref[...] = jnp.min(jnp.where(dists == m, kiota, K), axis=-1)


def _sc_gather(codebook_pad, idx_flat, N, D, DP):
    b_per_w = N // _NW
    mesh = plsc.VectorSubcoreMesh(core_axis_name="c", subcore_axis_name="s")

    @functools.partial(
        pl.kernel,
        mesh=mesh,
        out_type=jax.ShapeDtypeStruct((N, DP), jnp.float32),
        scratch_types=[
            pltpu.VMEM((b_per_w,), jnp.int32),
            pltpu.VMEM((b_per_w, DP), jnp.float32),
            pltpu.SemaphoreType.DMA,
        ],
    )
    def gk(table_hbm, idx_hbm, out_hbm, idx_v, rows_v, sem):
        wid = lax.axis_index("s") * _NC + lax.axis_index("c")
        base = wid * b_per_w
        pltpu.sync_copy(idx_hbm.at[pl.ds(base, b_per_w)], idx_v)
        pltpu.async_copy(table_hbm.at[idx_v], rows_v, sem).wait()
        pltpu.sync_copy(rows_v, out_hbm.at[pl.ds(base, b_per_w)])

    return gk(codebook_pad, idx_flat)


def kernel(z_e, codebook):
    B, T, D = z_e.shape
    K = codebook.shape[0]
    N = B * T
    z = z_e.reshape(N, D)
    cbT = codebook.T
    BT = 512

    idx_flat = pl.pallas_call(
        _vq_idx_body,
        grid=(N // BT,),
        in_specs=[
            pl.BlockSpec((BT, D), lambda i: (i, 0)),
            pl.BlockSpec((D, K), lambda i: (0, 0)),
            pl.BlockSpec((K, D), lambda i: (0, 0)),
        ],
        out_specs=pl.BlockSpec((BT,), lambda i: (i,)),
        out_shape=jax.ShapeDtypeStruct((N,), jnp.int32),
    )(z, cbT, codebook)

    DP = 128
    codebook_pad = jnp.pad(codebook, ((0, 0), (0, DP - D)))
    zq_pad = _sc_gather(codebook_pad, idx_flat, N, D, DP)
    return idx_flat.reshape(B, T), zq_pad[:, :D].reshape(B, T, D)


# speedup vs baseline: 1.0885x; 1.0885x over previous
import functools

import jax
import jax.numpy as jnp
from jax import lax
from jax.experimental import pallas as pl
from jax.experimental.pallas import tpu as pltpu
from jax.experimental.pallas import tpu_sc as plsc

_NC, _NS = 2, 16
_NW = _NC * _NS


def _vq_idx_body(z_ref, cb_ref, idx_ref):
    zb = z_ref[...]
    cb = cb_ref[...]
    K = cb.shape[0]
    BT = zb.shape[0]
    cross = lax.dot_general(
        zb, cb, (((1,), (1,)), ((), ())),
        preferred_element_type=jnp.float32)
    z_sq = jnp.sum(zb * zb, axis=-1, keepdims=True)
    e_sq = jnp.sum(cb * cb, axis=-1)
    dists = z_sq - 2.0 * cross + e_sq[None, :]
    m = jnp.min(dists, axis=-1, keepdims=True)
    kiota = lax.broadcasted_iota(jnp.int32, (BT, K), 1)
    idx_ref[...] = jnp.min(jnp.where(dists == m, kiota, K), axis=-1)


def _sc_gather(codebook, idx_flat, N, D):
    b_per_w = N // _NW
    mesh = plsc.VectorSubcoreMesh(core_axis_name="c", subcore_axis_name="s")

    @functools.partial(
        pl.kernel,
        mesh=mesh,
        out_type=jax.ShapeDtypeStruct((N, D), jnp.float32),
        scratch_types=[
            pltpu.VMEM((b_per_w,), jnp.int32),
            pltpu.VMEM((b_per_w, D), jnp.float32),
            pltpu.SemaphoreType.DMA,
        ],
        compiler_params=pltpu.CompilerParams(use_tc_tiling_on_sc=False),
    )
    def gk(table_hbm, idx_hbm, out_hbm, idx_v, rows_v, sem):
        wid = lax.axis_index("s") * _NC + lax.axis_index("c")
        base = wid * b_per_w
        pltpu.sync_copy(idx_hbm.at[pl.ds(base, b_per_w)], idx_v)
        pltpu.async_copy(table_hbm.at[idx_v], rows_v, sem).wait()
        pltpu.sync_copy(rows_v, out_hbm.at[pl.ds(base, b_per_w)])

    return gk(codebook, idx_flat)


def kernel(z_e, codebook):
    B, T, D = z_e.shape
    K = codebook.shape[0]
    N = B * T
    z = z_e.reshape(N, D)
    BT = 1024

    idx_flat = pl.pallas_call(
        _vq_idx_body,
        grid=(N // BT,),
        in_specs=[
            pl.BlockSpec((BT, D), lambda i: (i, 0)),
            pl.BlockSpec((K, D), lambda i: (0, 0)),
        ],
        out_specs=pl.BlockSpec((BT,), lambda i: (i,)),
        out_shape=jax.ShapeDtypeStruct((N,), jnp.int32),
    )(z, codebook)

    zq_flat = _sc_gather(codebook, idx_flat, N, D)
    return idx_flat.reshape(B, T), zq_flat.reshape(B, T, D)


# speedup vs baseline: 1.1925x; 1.0955x over previous
import functools

import jax
import jax.numpy as jnp
from jax import lax
from jax.experimental import pallas as pl
from jax.experimental.pallas import tpu as pltpu
from jax.experimental.pallas import tpu_sc as plsc

_NC, _NS = 2, 16
_NW = _NC * _NS


def _vq_idx_body(z_ref, cb_ref, idx_ref):
    zb = z_ref[...].reshape(-1, z_ref.shape[-1])
    cb = cb_ref[...]
    K = cb.shape[0]
    BT = zb.shape[0]
    cross = lax.dot_general(
        zb, cb, (((1,), (1,)), ((), ())),
        preferred_element_type=jnp.float32)
    z_sq = jnp.sum(zb * zb, axis=-1, keepdims=True)
    e_sq = jnp.sum(cb * cb, axis=-1)
    C = 128
    TB = 64
    G = K // C
    liota = lax.broadcasted_iota(jnp.int32, (TB, C), 1)
    for t in range(BT // TB):
        zs = z_sq[t * TB:(t + 1) * TB, :]
        bd = zs - 2.0 * cross[t * TB:(t + 1) * TB, :C] + e_sq[None, :C]
        bg = jnp.zeros((TB, C), jnp.int32)
        for g in range(1, G):
            d = (zs - 2.0 * cross[t * TB:(t + 1) * TB, g * C:(g + 1) * C]
                 + e_sq[None, g * C:(g + 1) * C])
            lt = d < bd
            bd = jnp.where(lt, d, bd)
            bg = jnp.where(lt, g, bg)
        m = jnp.min(bd, axis=-1, keepdims=True)
        cand = jnp.where(bd == m, bg * C + liota, K)
        idx_ref[0, 0, pl.ds(t * TB, TB)] = jnp.min(cand, axis=-1)


def _sc_gather(codebook, idx_flat, N, D):
    b_per_w = N // _NW
    mesh = plsc.VectorSubcoreMesh(core_axis_name="c", subcore_axis_name="s")

    @functools.partial(
        pl.kernel,
        mesh=mesh,
        out_type=jax.ShapeDtypeStruct((N, D), jnp.float32),
        scratch_types=[
            pltpu.VMEM((b_per_w,), jnp.int32),
            pltpu.VMEM((b_per_w, D), jnp.float32),
            pltpu.SemaphoreType.DMA,
        ],
        compiler_params=pltpu.CompilerParams(use_tc_tiling_on_sc=False),
    )
    def gk(table_hbm, idx_hbm, out_hbm, idx_v, rows_v, sem):
        wid = lax.axis_index("s") * _NC + lax.axis_index("c")
        base = wid * b_per_w
        pltpu.sync_copy(idx_hbm.at[pl.ds(base, b_per_w)], idx_v)
        pltpu.async_copy(table_hbm.at[idx_v], rows_v, sem).wait()
        pltpu.sync_copy(rows_v, out_hbm.at[pl.ds(base, b_per_w)])

    return gk(codebook, idx_flat)


def kernel(z_e, codebook):
    B, T, D = z_e.shape
    K = codebook.shape[0]
    N = B * T
    BB = 2
    BT = BB * T
    G = B // BB

    idx3 = pl.pallas_call(
        _vq_idx_body,
        grid=(G,),
        in_specs=[
            pl.BlockSpec((BB, T, D), lambda i: (i, 0, 0)),
            pl.BlockSpec((K, D), lambda i: (0, 0)),
        ],
        out_specs=pl.BlockSpec((1, 1, BT), lambda i: (i, 0, 0)),
        out_shape=jax.ShapeDtypeStruct((G, 1, BT), jnp.int32),
    )(z_e, codebook)
    idx_flat = idx3.reshape(N)

    zq_flat = _sc_gather(codebook, idx_flat, N, D)
    return idx_flat.reshape(B, T), zq_flat.reshape(B, T, D)
